# baseline (speedup 1.0000x reference)
import functools

import jax
import jax.numpy as jnp
from jax.experimental import pallas as pl
from jax.experimental.pallas import tpu as pltpu

_TOPK = 2


def _grouped_ffn_body(tile_r, lo_r, hi_r, te_r, xg_ref, gw_ref, uw_ref, dw_ref,
                      out_ref, *, m):
    i = pl.program_id(0)
    s = pl.program_id(1)
    tile = tile_r[s]
    lo = lo_r[s]
    hi = hi_r[s]

    xb = xg_ref[pl.ds(tile * m, m), :]
    gw = gw_ref[0].astype(jnp.bfloat16)
    uw = uw_ref[0].astype(jnp.bfloat16)
    dw = dw_ref[0].astype(jnp.bfloat16)

    dn = (((1,), (1,)), ((), ()))
    g = jax.lax.dot_general(xb, gw, dn, preferred_element_type=jnp.float32)
    u = jax.lax.dot_general(xb, uw, dn, preferred_element_type=jnp.float32)
    h = (jax.nn.silu(g) * u).astype(jnp.bfloat16)

    contrib = jax.lax.dot_general(h, dw, dn, preferred_element_type=jnp.float32)

    row = jax.lax.broadcasted_iota(jnp.int32, (m, 1), 0)
    contrib = jnp.where((row >= lo) & (row < hi), contrib, 0.0)

    @pl.when((i == 0) & (lo == 0))
    def _set():
        out_ref[pl.ds(tile * m, m), :] = contrib

    @pl.when((i > 0) | (lo > 0))
    def _acc():
        out_ref[pl.ds(tile * m, m), :] += contrib


def kernel(x, router_w, router_b, gate_w, up_w, down_w):
    B, S, D = x.shape
    E, DI, _ = gate_w.shape
    N = B * S
    P = N * _TOPK
    xf = x.reshape(N, D)

    logits = xf @ router_w.T + router_b
    probs = jax.nn.softmax(logits, axis=-1)
    topk_p, topk_i = jax.lax.top_k(probs, _TOPK)
    topk_p = topk_p / jnp.sum(topk_p, axis=-1, keepdims=True)

    ei = topk_i.reshape(P).astype(jnp.int32)
    wv = topk_p.reshape(P)
    tok = (jnp.arange(P, dtype=jnp.int32) // _TOPK)

    M = 256
    NSEG = P // M + E
    order = jnp.argsort(ei)
    stok = tok[order]
    counts = jnp.bincount(ei, length=E)
    gend = jnp.cumsum(counts).astype(jnp.int32)
    gstart = jnp.concatenate([jnp.zeros((1,), jnp.int32), gend[:-1]])

    bps = jnp.sort(
        jnp.concatenate([jnp.arange(P // M, dtype=jnp.int32) * M, gstart])
    )
    ends = jnp.concatenate([bps[1:], jnp.full((1,), P, jnp.int32)])
    seg_tile = bps // M
    seg_lo = bps - seg_tile * M
    seg_hi = ends - seg_tile * M
    seg_te = jnp.minimum(
        jnp.searchsorted(gend, bps, side="right"), E - 1
    ).astype(jnp.int32)

    xg = jnp.take(xf.astype(jnp.bfloat16), stok, axis=0)

    Ki = min(512, DI)
    NI = DI // Ki

    grid_spec = pltpu.PrefetchScalarGridSpec(
        num_scalar_prefetch=4,
        grid=(NI, NSEG),
        in_specs=[
            pl.BlockSpec((P, D), lambda i, s, t_r, l_r, h_r, e_r: (0, 0)),
            pl.BlockSpec((1, Ki, D), lambda i, s, t_r, l_r, h_r, e_r: (e_r[s], i, 0)),
            pl.BlockSpec((1, Ki, D), lambda i, s, t_r, l_r, h_r, e_r: (e_r[s], i, 0)),
            pl.BlockSpec((1, D, Ki), lambda i, s, t_r, l_r, h_r, e_r: (e_r[s], 0, i)),
        ],
        out_specs=pl.BlockSpec((P, D), lambda i, s, t_r, l_r, h_r, e_r: (0, 0)),
    )

    yg = pl.pallas_call(
        functools.partial(_grouped_ffn_body, m=M),
        grid_spec=grid_spec,
        out_shape=jax.ShapeDtypeStruct((P, D), jnp.float32),
        compiler_params=pltpu.CompilerParams(vmem_limit_bytes=67108864),
    )(seg_tile, seg_lo, seg_hi, seg_te, xg, gate_w, up_w, down_w)

    posp = jnp.zeros((P,), jnp.int32).at[order].set(jnp.arange(P, dtype=jnp.int32))
    y = (wv[:, None] * yg[posp]).reshape(N, _TOPK, D).sum(axis=1)
    return y.reshape(B, S, D)


# speedup vs baseline: 1.0787x; 1.0787x over previous
import functools

import jax
import jax.numpy as jnp
from jax import lax
from jax.experimental import pallas as pl
from jax.experimental.pallas import tpu as pltpu
from jax.experimental.pallas import tpu_sc as plsc

_TOPK = 2


def _sc_gather(table, idx, chunk=64):
    V, D = table.shape
    (P,) = idx.shape
    info = plsc.get_sparse_core_info()
    NW = info.num_cores * info.num_subcores
    per_w = P // NW
    n_chunks = per_w // chunk
    mesh = plsc.VectorSubcoreMesh(core_axis_name="c", subcore_axis_name="s")

    @functools.partial(
        pl.kernel,
        mesh=mesh,
        out_type=jax.ShapeDtypeStruct((P, D), table.dtype),
        scratch_types=[
            pltpu.VMEM((chunk,), jnp.int32),
            pltpu.VMEM((chunk, D), table.dtype),
            pltpu.SemaphoreType.DMA,
        ],
    )
    def k(table_hbm, idx_hbm, out_hbm, idx_v, rows_v, sem):
        wid = lax.axis_index("s") * info.num_cores + lax.axis_index("c")
        base = wid * per_w
        for ch in range(n_chunks):
            off = base + ch * chunk
            pltpu.sync_copy(idx_hbm.at[pl.ds(off, chunk)], idx_v)
            pltpu.async_copy(table_hbm.at[idx_v], rows_v, sem).wait()
            pltpu.sync_copy(rows_v, out_hbm.at[pl.ds(off, chunk)])

    return k(table, idx)


def _grouped_ffn_body(tile_r, lo_r, hi_r, te_r, xg_ref, gw_ref, uw_ref, dw_ref,
                      out_ref, *, m):
    i = pl.program_id(0)
    s = pl.program_id(1)
    tile = tile_r[s]
    lo = lo_r[s]
    hi = hi_r[s]

    xb = xg_ref[pl.ds(tile * m, m), :].astype(jnp.float32)
    gw = gw_ref[0]
    uw = uw_ref[0]
    dw = dw_ref[0]

    dn = (((1,), (1,)), ((), ()))
    g = jax.lax.dot_general(xb, gw, dn, preferred_element_type=jnp.float32)
    u = jax.lax.dot_general(xb, uw, dn, preferred_element_type=jnp.float32)
    h = jax.nn.silu(g) * u

    contrib = jax.lax.dot_general(h, dw, dn, preferred_element_type=jnp.float32)

    row = jax.lax.broadcasted_iota(jnp.int32, (m, 1), 0)
    contrib = jnp.where((row >= lo) & (row < hi), contrib, 0.0)

    @pl.when((i == 0) & (lo == 0))
    def _set():
        out_ref[pl.ds(tile * m, m), :] = contrib

    @pl.when((i > 0) | (lo > 0))
    def _acc():
        out_ref[pl.ds(tile * m, m), :] += contrib


def kernel(x, router_w, router_b, gate_w, up_w, down_w):
    B, S, D = x.shape
    E, DI, _ = gate_w.shape
    N = B * S
    P = N * _TOPK
    xf = x.reshape(N, D)

    logits = xf @ router_w.T + router_b
    probs = jax.nn.softmax(logits, axis=-1)
    topk_p, topk_i = jax.lax.top_k(probs, _TOPK)
    topk_p = topk_p / jnp.sum(topk_p, axis=-1, keepdims=True)

    ei = topk_i.reshape(P).astype(jnp.int32)
    wv = topk_p.reshape(P)
    tok = (jnp.arange(P, dtype=jnp.int32) // _TOPK)

    M = 256
    NSEG = P // M + E
    order = jnp.argsort(ei)
    stok = tok[order]
    counts = jnp.bincount(ei, length=E)
    gend = jnp.cumsum(counts).astype(jnp.int32)
    gstart = jnp.concatenate([jnp.zeros((1,), jnp.int32), gend[:-1]])

    bps = jnp.sort(
        jnp.concatenate([jnp.arange(P // M, dtype=jnp.int32) * M, gstart])
    )
    ends = jnp.concatenate([bps[1:], jnp.full((1,), P, jnp.int32)])
    seg_tile = bps // M
    seg_lo = bps - seg_tile * M
    seg_hi = ends - seg_tile * M
    seg_te = jnp.minimum(
        jnp.searchsorted(gend, bps, side="right"), E - 1
    ).astype(jnp.int32)

    xg = _sc_gather(xf, stok).astype(jnp.bfloat16)

    Ki = min(512, DI)
    NI = DI // Ki

    grid_spec = pltpu.PrefetchScalarGridSpec(
        num_scalar_prefetch=4,
        grid=(NI, NSEG),
        in_specs=[
            pl.BlockSpec((P, D), lambda i, s, t_r, l_r, h_r, e_r: (0, 0)),
            pl.BlockSpec((1, Ki, D), lambda i, s, t_r, l_r, h_r, e_r: (e_r[s], i, 0)),
            pl.BlockSpec((1, Ki, D), lambda i, s, t_r, l_r, h_r, e_r: (e_r[s], i, 0)),
            pl.BlockSpec((1, D, Ki), lambda i, s, t_r, l_r, h_r, e_r: (e_r[s], 0, i)),
        ],
        out_specs=pl.BlockSpec((P, D), lambda i, s, t_r, l_r, h_r, e_r: (0, 0)),
    )

    yg = pl.pallas_call(
        functools.partial(_grouped_ffn_body, m=M),
        grid_spec=grid_spec,
        out_shape=jax.ShapeDtypeStruct((P, D), jnp.float32),
        compiler_params=pltpu.CompilerParams(vmem_limit_bytes=67108864),
    )(seg_tile, seg_lo, seg_hi, seg_te, xg, gate_w, up_w, down_w)

    posp = jnp.zeros((P,), jnp.int32).at[order].set(jnp.arange(P, dtype=jnp.int32))
    y = (wv[:, None] * yg[posp]).reshape(N, _TOPK, D).sum(axis=1)
    return y.reshape(B, S, D)
